# Initial kernel scaffold; baseline (speedup 1.0000x reference)
import functools

import jax
import jax.numpy as jnp
from jax import lax
from jax.experimental import pallas as pl
from jax.experimental.pallas import tpu as pltpu
from jax.experimental.pallas import tpu_sc as plsc

_NUM_MOODS = 100000
_EMBED_DIM = 32
_NUM_EMBEDDINGS = _NUM_MOODS + 1

_L = 16
_NW = 32
_IDXW = 128


def _make_gather(batch: int, chunk: int):
    assert batch % (_NW * chunk) == 0
    assert chunk % _IDXW == 0
    b_per_w = batch // _NW
    n_chunks = b_per_w // chunk
    k = chunk // _IDXW

    mesh = plsc.VectorSubcoreMesh(core_axis_name="c", subcore_axis_name="s")

    @functools.partial(
        pl.kernel,
        mesh=mesh,
        out_type=jax.ShapeDtypeStruct((batch, _EMBED_DIM), jnp.float32),
        scratch_types=[
            pltpu.VMEM((k, _IDXW), jnp.int32),
            pltpu.VMEM((chunk, _EMBED_DIM), jnp.float32),
            pltpu.SemaphoreType.DMA,
        ],
    )
    def gather_kernel(ids_hbm, table_hbm, out_hbm, idx_v, rows_v, sem):
        wid = lax.axis_index("s") * 2 + lax.axis_index("c")
        idx_row_base = wid * (b_per_w // _IDXW)
        out_base = wid * b_per_w

        def chunk_body(g, carry):
            pltpu.sync_copy(
                ids_hbm.at[pl.ds(idx_row_base + g * k, k)],
                idx_v,
            )
            for j in range(k):
                for i in range(_IDXW // _L):
                    v = idx_v[j, pl.ds(i * _L, _L)]
                    v = jnp.minimum(jnp.maximum(v, 0), _NUM_EMBEDDINGS - 1)
                    idx_v[j, pl.ds(i * _L, _L)] = v
            copies = []
            for j in range(k):
                copies.append(
                    pltpu.async_copy(
                        table_hbm.at[idx_v.at[j]],
                        rows_v.at[pl.ds(j * _IDXW, _IDXW)],
                        sem,
                    )
                )
            for c in copies:
                c.wait()
            pltpu.sync_copy(
                rows_v,
                out_hbm.at[pl.ds(out_base + g * chunk, chunk)],
            )
            return carry

        lax.fori_loop(0, n_chunks, chunk_body, 0)

    return gather_kernel


def kernel(mood_ids, table):
    b0, s = mood_ids.shape
    batch = b0 * s
    ids2d = mood_ids.astype(jnp.int32).reshape(batch // _IDXW, _IDXW)
    out = _make_gather(batch, 1280)(ids2d, table)
    return out.reshape(b0, s, _EMBED_DIM)


# speedup vs baseline: 2.9615x; 2.9615x over previous
import functools

import jax
import jax.numpy as jnp
from jax import lax
from jax.experimental import pallas as pl
from jax.experimental.pallas import tpu as pltpu
from jax.experimental.pallas import tpu_sc as plsc

_NUM_MOODS = 100000
_EMBED_DIM = 32
_NUM_EMBEDDINGS = _NUM_MOODS + 1

_L = 16
_NW = 32
_IDXW = 128


def _make_gather(batch: int, chunk: int):
    assert batch % (_NW * chunk) == 0
    assert chunk % _IDXW == 0
    b_per_w = batch // _NW
    n_chunks = b_per_w // chunk
    k = chunk // _IDXW

    mesh = plsc.VectorSubcoreMesh(core_axis_name="c", subcore_axis_name="s")

    @functools.partial(
        pl.kernel,
        mesh=mesh,
        out_type=jax.ShapeDtypeStruct((batch, _EMBED_DIM), jnp.float32),
        scratch_types=[
            pltpu.VMEM((chunk,), jnp.int32),
            pltpu.VMEM((chunk, _EMBED_DIM), jnp.float32),
            pltpu.SemaphoreType.DMA,
        ],
        compiler_params=pltpu.CompilerParams(use_tc_tiling_on_sc=False),
    )
    def gather_kernel(ids_hbm, table_hbm, out_hbm, idx_v, rows_v, sem):
        wid = lax.axis_index("s") * 2 + lax.axis_index("c")
        out_base = wid * b_per_w

        def chunk_body(g, carry):
            pltpu.sync_copy(
                ids_hbm.at[pl.ds(out_base + g * chunk, chunk)],
                idx_v,
            )
            for t in range(chunk // _L):
                v = idx_v[pl.ds(t * _L, _L)]
                v = jnp.minimum(jnp.maximum(v, 0), _NUM_EMBEDDINGS - 1)
                idx_v[pl.ds(t * _L, _L)] = v
            copies = []
            for j in range(k):
                copies.append(
                    pltpu.async_copy(
                        table_hbm.at[idx_v.at[pl.ds(j * _IDXW, _IDXW)]],
                        rows_v.at[pl.ds(j * _IDXW, _IDXW)],
                        sem,
                    )
                )
            for c in copies:
                c.wait()
            pltpu.sync_copy(
                rows_v,
                out_hbm.at[pl.ds(out_base + g * chunk, chunk)],
            )
            return carry

        lax.fori_loop(0, n_chunks, chunk_body, 0)

    return gather_kernel


def kernel(mood_ids, table):
    b0, s = mood_ids.shape
    batch = b0 * s
    ids_flat = mood_ids.astype(jnp.int32).reshape(batch)
    out = _make_gather(batch, 1280)(ids_flat, table)
    return out.reshape(b0, s, _EMBED_DIM)


# speedup vs baseline: 3.0106x; 1.0166x over previous
import functools

import jax
import jax.numpy as jnp
from jax import lax
from jax.experimental import pallas as pl
from jax.experimental.pallas import tpu as pltpu
from jax.experimental.pallas import tpu_sc as plsc

_NUM_MOODS = 100000
_EMBED_DIM = 32
_NUM_EMBEDDINGS = _NUM_MOODS + 1

_L = 16
_NW = 32
_IDXW = 128
_CLAMP_UNROLL = 8


def _make_gather(batch: int, chunk: int):
    assert batch % (_NW * chunk) == 0
    assert chunk % (_CLAMP_UNROLL * _L) == 0 and chunk % _IDXW == 0
    b_per_w = batch // _NW
    n_chunks = b_per_w // chunk
    assert n_chunks >= 3
    k = chunk // _IDXW

    mesh = plsc.VectorSubcoreMesh(core_axis_name="c", subcore_axis_name="s")

    @functools.partial(
        pl.kernel,
        mesh=mesh,
        out_type=jax.ShapeDtypeStruct((batch, _EMBED_DIM), jnp.float32),
        scratch_types=[
            [pltpu.VMEM((chunk,), jnp.int32)] * 3,
            [pltpu.VMEM((chunk, _EMBED_DIM), jnp.float32)] * 2,
            [pltpu.SemaphoreType.DMA] * 3,
            [pltpu.SemaphoreType.DMA] * 2,
            [pltpu.SemaphoreType.DMA] * 2,
        ],
        compiler_params=pltpu.CompilerParams(use_tc_tiling_on_sc=False),
    )
    def gather_kernel(ids_hbm, table_hbm, out_hbm, idx_v, rows_v, isem, gsem,
                      osem):
        wid = lax.axis_index("s") * 2 + lax.axis_index("c")
        base = wid * b_per_w

        def load_idx(g, q):
            return pltpu.async_copy(
                ids_hbm.at[pl.ds(base + g * chunk, chunk)],
                idx_v[q], isem[q])

        def clamp(q):
            def body(t, carry):
                for u in range(_CLAMP_UNROLL):
                    sl = pl.ds(t * (_CLAMP_UNROLL * _L) + u * _L, _L)
                    v = idx_v[q][sl]
                    idx_v[q][sl] = jnp.minimum(
                        jnp.maximum(v, 0), _NUM_EMBEDDINGS - 1)
                return carry
            lax.fori_loop(0, chunk // (_CLAMP_UNROLL * _L), body, 0)

        ih = [None, None, None]
        gh = [None, None]
        oh = [None, None]

        ih[0] = load_idx(0, 0)
        ih[1] = load_idx(1, 1)

        for g in range(n_chunks):
            p = g & 1
            q = g % 3
            ih[q].wait()
            clamp(q)
            if g >= 2:
                oh[p].wait()
            gh[p] = [
                pltpu.async_copy(
                    table_hbm.at[idx_v[q].at[pl.ds(j * _IDXW, _IDXW)]],
                    rows_v[p].at[pl.ds(j * _IDXW, _IDXW)],
                    gsem[p])
                for j in range(k)
            ]
            if g >= 1:
                for c in gh[1 - p]:
                    c.wait()
                oh[1 - p] = pltpu.async_copy(
                    rows_v[1 - p],
                    out_hbm.at[pl.ds(base + (g - 1) * chunk, chunk)],
                    osem[1 - p])
            if g + 2 < n_chunks:
                ih[(g + 2) % 3] = load_idx(g + 2, (g + 2) % 3)

        pl_ = (n_chunks - 1) & 1
        for c in gh[pl_]:
            c.wait()
        oh[pl_] = pltpu.async_copy(
            rows_v[pl_],
            out_hbm.at[pl.ds(base + (n_chunks - 1) * chunk, chunk)],
            osem[pl_])
        oh[0].wait()
        oh[1].wait()

    return gather_kernel


def kernel(mood_ids, table):
    b0, s = mood_ids.shape
    batch = b0 * s
    ids_flat = mood_ids.astype(jnp.int32).reshape(batch)
    out = _make_gather(batch, 1280)(ids_flat, table)
    return out.reshape(b0, s, _EMBED_DIM)


# speedup vs baseline: 5.2611x; 1.7475x over previous
import functools

import jax
import jax.numpy as jnp
from jax import lax
from jax.experimental import pallas as pl
from jax.experimental.pallas import tpu as pltpu
from jax.experimental.pallas import tpu_sc as plsc

_NUM_MOODS = 100000
_EMBED_DIM = 32
_NUM_EMBEDDINGS = _NUM_MOODS + 1

_L = 16
_NW = 32
_IDXW = 128


def _make_kernel(n_b: int, n_s: int):
    assert n_b % (_NW * _IDXW) == 0
    bpw = n_b // _NW
    kk = bpw // _IDXW
    n_ct = n_b // _IDXW
    n_rt = _EMBED_DIM // 8

    mesh = plsc.VectorSubcoreMesh(core_axis_name="c", subcore_axis_name="s")

    @functools.partial(
        pl.kernel,
        mesh=mesh,
        out_type=jax.ShapeDtypeStruct((n_s, n_rt, n_ct, 8, _IDXW),
                                      jnp.float32),
        scratch_types=[
            pltpu.VMEM((bpw * n_s,), jnp.int32),
            [pltpu.VMEM((bpw,), jnp.int32)] * 2,
            [pltpu.VMEM((bpw, _EMBED_DIM), jnp.float32)] * 2,
            [pltpu.VMEM((n_rt, kk, 8, _IDXW), jnp.float32)] * 2,
            pltpu.SemaphoreType.DMA,
            [pltpu.SemaphoreType.DMA] * 2,
            [pltpu.SemaphoreType.DMA] * 2,
        ],
        compiler_params=pltpu.CompilerParams(
            use_tc_tiling_on_sc=False, needs_layout_passes=False),
    )
    def gather_kernel(ids_hbm, table_hbm, out_hbm, blk, idx_v, rows_v, stage_v,
                      bsem, gsem, osem):
        wid = lax.axis_index("s") * 2 + lax.axis_index("c")
        ct0 = wid * kk

        pltpu.async_copy(
            ids_hbm.at[pl.ds(wid * (bpw * n_s), bpw * n_s)], blk, bsem
        ).wait()

        iota = lax.iota(jnp.int32, _L)
        iota_ns = iota * n_s
        iota_d = iota * _EMBED_DIM

        def extract_clamp(s, p):
            def body(c, carry):
                v = plsc.load_gather(blk, [iota_ns + (s + c * (_L * n_s))])
                v = jnp.minimum(jnp.maximum(v, 0), _NUM_EMBEDDINGS - 1)
                idx_v[p][pl.ds(c * _L, _L)] = v
                return carry
            lax.fori_loop(0, bpw // _L, body, 0)

        def fire_gathers(p):
            return [
                pltpu.async_copy(
                    table_hbm.at[idx_v[p].at[pl.ds(j * _IDXW, _IDXW)]],
                    rows_v[p].at[pl.ds(j * _IDXW, _IDXW)],
                    gsem[p])
                for j in range(kk)
            ]

        def transpose_fill(p):
            def body(rc, carry):
                r = rc // kk
                c = rc % kk
                for d in range(8):
                    col = jnp.full((_L,), r * 8 + d, jnp.int32)
                    for t in range(_IDXW // _L):
                        rowv = iota + (c * _IDXW + t * _L)
                        vec = plsc.load_gather(rows_v[p], [rowv, col])
                        stage_v[p][r, c, d, pl.ds(t * _L, _L)] = vec
                return carry
            lax.fori_loop(0, n_rt * kk, body, 0)

        def fire_stores(s, p):
            return [
                pltpu.async_copy(
                    stage_v[p].at[r],
                    out_hbm.at[s, r, pl.ds(ct0, kk)],
                    osem[p])
                for r in range(n_rt)
            ]

        def wait_gathers(p):
            for j in range(kk):
                pltpu.make_async_copy(
                    table_hbm.at[idx_v[p].at[pl.ds(j * _IDXW, _IDXW)]],
                    rows_v[p].at[pl.ds(j * _IDXW, _IDXW)],
                    gsem[p]).wait()

        def wait_stores(s, p):
            for r in range(n_rt):
                pltpu.make_async_copy(
                    stage_v[p].at[r],
                    out_hbm.at[s, r, pl.ds(ct0, kk)],
                    osem[p]).wait()

        extract_clamp(0, 0)
        fire_gathers(0)
        extract_clamp(1, 1)
        wait_gathers(0)
        fire_gathers(1)
        transpose_fill(0)
        fire_stores(0, 0)
        extract_clamp(2, 0)
        wait_gathers(1)
        fire_gathers(0)
        transpose_fill(1)
        fire_stores(1, 1)

        def pair_body(t, carry):
            s0 = 2 * t
            s1 = s0 + 1
            extract_clamp(s1, 1)
            wait_gathers(0)
            fire_gathers(1)
            wait_stores(s0 - 2, 0)
            transpose_fill(0)
            fire_stores(s0, 0)
            extract_clamp(jnp.minimum(s0 + 2, n_s - 1), 0)
            wait_gathers(1)
            fire_gathers(0)
            wait_stores(s1 - 2, 1)
            transpose_fill(1)
            fire_stores(s1, 1)
            return carry

        lax.fori_loop(1, n_s // 2, pair_body, 0)

        wait_gathers(0)
        wait_stores(n_s - 2, 0)
        wait_stores(n_s - 1, 1)

    return gather_kernel


def kernel(mood_ids, table):
    n_b, n_s = mood_ids.shape
    ids_flat = mood_ids.astype(jnp.int32).reshape(n_b * n_s)
    out5 = _make_kernel(n_b, n_s)(ids_flat, table)
    out = jnp.transpose(out5, (2, 4, 0, 1, 3)).reshape(n_b, n_s, _EMBED_DIM)
    return out


# speedup vs baseline: 6.4577x; 1.2275x over previous
import functools

import jax
import jax.numpy as jnp
from jax import lax
from jax.experimental import pallas as pl
from jax.experimental.pallas import tpu as pltpu
from jax.experimental.pallas import tpu_sc as plsc

_NUM_MOODS = 100000
_EMBED_DIM = 32
_NUM_EMBEDDINGS = _NUM_MOODS + 1

_L = 16
_NW = 32
_IDXW = 128


def _make_kernel(n_b: int, n_s: int):
    assert n_b % (_NW * _IDXW) == 0
    bpw = n_b // _NW
    kk = bpw // _IDXW
    n_ct = n_b // _IDXW
    n_rt = _EMBED_DIM // 8

    mesh = plsc.VectorSubcoreMesh(core_axis_name="c", subcore_axis_name="s")

    @functools.partial(
        pl.kernel,
        mesh=mesh,
        out_type=jax.ShapeDtypeStruct((n_s, n_rt, n_ct, 8, _IDXW),
                                      jnp.float32),
        scratch_types=[
            pltpu.VMEM((bpw * n_s,), jnp.int32),
            [pltpu.VMEM((bpw,), jnp.int32)] * 2,
            [pltpu.VMEM((bpw, _EMBED_DIM), jnp.float32)] * 2,
            [pltpu.VMEM((n_rt, kk, 8, _IDXW), jnp.float32)] * 2,
            pltpu.SemaphoreType.DMA,
            [pltpu.SemaphoreType.DMA] * 2,
            [pltpu.SemaphoreType.DMA] * 2,
        ],
        compiler_params=pltpu.CompilerParams(
            use_tc_tiling_on_sc=False, needs_layout_passes=False),
    )
    def gather_kernel(ids_hbm, table_hbm, out_hbm, blk, idx_v, rows_v, stage_v,
                      bsem, gsem, osem):
        wid = lax.axis_index("s") * 2 + lax.axis_index("c")
        ct0 = wid * kk

        pltpu.async_copy(
            ids_hbm.at[pl.ds(wid * (bpw * n_s), bpw * n_s)], blk, bsem
        ).wait()

        iota = lax.iota(jnp.int32, _L)
        iota_ns = iota * n_s
        iota_d = iota * _EMBED_DIM

        def extract_clamp(s, p):
            def body(c, carry):
                v = plsc.load_gather(blk, [iota_ns + (s + c * (_L * n_s))])
                v = jnp.minimum(jnp.maximum(v, 0), _NUM_EMBEDDINGS - 1)
                idx_v[p][pl.ds(c * _L, _L)] = v
                return carry
            lax.fori_loop(0, bpw // _L, body, 0)

        def fire_gathers(p):
            return [
                pltpu.async_copy(
                    table_hbm.at[idx_v[p].at[pl.ds(j * _IDXW, _IDXW)]],
                    rows_v[p].at[pl.ds(j * _IDXW, _IDXW)],
                    gsem[p])
                for j in range(kk)
            ]

        rowt = [iota + t * _L for t in range(_IDXW // _L)]

        def transpose_fill(p):
            def body(rc, carry):
                r = rc // kk
                c = rc % kk
                cbase = jnp.full((_L,), c * _IDXW, jnp.int32)
                rows8 = [rv + cbase for rv in rowt]
                for d0 in range(0, 8, 2):
                    vecs = []
                    for d in (d0, d0 + 1):
                        col = jnp.full((_L,), r * 8 + d, jnp.int32)
                        for t in range(_IDXW // _L):
                            vecs.append(
                                plsc.load_gather(rows_v[p], [rows8[t], col]))
                    for i, vec in enumerate(vecs):
                        d, t = divmod(i, _IDXW // _L)
                        stage_v[p][r, c, d0 + d, pl.ds(t * _L, _L)] = vec
                return carry
            lax.fori_loop(0, n_rt * kk, body, 0)

        def fire_stores(s, p):
            return [
                pltpu.async_copy(
                    stage_v[p].at[r],
                    out_hbm.at[s, r, pl.ds(ct0, kk)],
                    osem[p])
                for r in range(n_rt)
            ]

        def wait_gathers(p):
            for j in range(kk):
                pltpu.make_async_copy(
                    table_hbm.at[idx_v[p].at[pl.ds(j * _IDXW, _IDXW)]],
                    rows_v[p].at[pl.ds(j * _IDXW, _IDXW)],
                    gsem[p]).wait()

        def wait_stores(s, p):
            for r in range(n_rt):
                pltpu.make_async_copy(
                    stage_v[p].at[r],
                    out_hbm.at[s, r, pl.ds(ct0, kk)],
                    osem[p]).wait()

        extract_clamp(0, 0)
        fire_gathers(0)
        extract_clamp(1, 1)
        wait_gathers(0)
        fire_gathers(1)
        transpose_fill(0)
        fire_stores(0, 0)
        extract_clamp(2, 0)
        wait_gathers(1)
        fire_gathers(0)
        transpose_fill(1)
        fire_stores(1, 1)

        def pair_body(t, carry):
            s0 = 2 * t
            s1 = s0 + 1
            extract_clamp(s1, 1)
            wait_gathers(0)
            fire_gathers(1)
            wait_stores(s0 - 2, 0)
            transpose_fill(0)
            fire_stores(s0, 0)
            extract_clamp(jnp.minimum(s0 + 2, n_s - 1), 0)
            wait_gathers(1)
            fire_gathers(0)
            wait_stores(s1 - 2, 1)
            transpose_fill(1)
            fire_stores(s1, 1)
            return carry

        lax.fori_loop(1, n_s // 2, pair_body, 0)

        wait_gathers(0)
        wait_stores(n_s - 2, 0)
        wait_stores(n_s - 1, 1)

    return gather_kernel


def kernel(mood_ids, table):
    n_b, n_s = mood_ids.shape
    ids_flat = mood_ids.astype(jnp.int32).reshape(n_b * n_s)
    out5 = _make_kernel(n_b, n_s)(ids_flat, table)
    out = jnp.transpose(out5, (2, 4, 0, 1, 3)).reshape(n_b, n_s, _EMBED_DIM)
    return out


# speedup vs baseline: 6.5585x; 1.0156x over previous
import functools

import jax
import jax.numpy as jnp
from jax import lax
from jax.experimental import pallas as pl
from jax.experimental.pallas import tpu as pltpu
from jax.experimental.pallas import tpu_sc as plsc

_NUM_MOODS = 100000
_EMBED_DIM = 32
_NUM_EMBEDDINGS = _NUM_MOODS + 1

_L = 16
_NW = 32
_IDXW = 128


def _make_kernel(n_b: int, n_s: int):
    assert n_b % (_NW * _IDXW) == 0
    bpw = n_b // _NW
    kk = bpw // _IDXW
    n_ct = n_b // _IDXW
    n_rt = _EMBED_DIM // 8

    mesh = plsc.VectorSubcoreMesh(core_axis_name="c", subcore_axis_name="s")

    @functools.partial(
        pl.kernel,
        mesh=mesh,
        out_type=jax.ShapeDtypeStruct((n_s, n_rt, n_ct, 8, _IDXW),
                                      jnp.float32),
        scratch_types=[
            pltpu.VMEM((bpw * n_s,), jnp.int32),
            [pltpu.VMEM((bpw,), jnp.int32)] * 2,
            [pltpu.VMEM((bpw, _EMBED_DIM), jnp.float32)] * 2,
            [pltpu.VMEM((n_rt, kk, 8, _IDXW), jnp.float32)] * 2,
            pltpu.SemaphoreType.DMA,
            [pltpu.SemaphoreType.DMA] * 2,
            [pltpu.SemaphoreType.DMA] * 2,
        ],
        compiler_params=pltpu.CompilerParams(
            use_tc_tiling_on_sc=False, needs_layout_passes=False),
    )
    def gather_kernel(ids_hbm, table_hbm, out_hbm, blk, idx_v, rows_v, stage_v,
                      bsem, gsem, osem):
        wid = lax.axis_index("s") * 2 + lax.axis_index("c")
        ct0 = wid * kk

        pltpu.async_copy(
            ids_hbm.at[pl.ds(wid * (bpw * n_s), bpw * n_s)], blk, bsem
        ).wait()

        iota = lax.iota(jnp.int32, _L)
        iota_ns = iota * n_s
        iota_d = iota * _EMBED_DIM

        def extract_clamp(s, p):
            def body(c, carry):
                v = plsc.load_gather(blk, [iota_ns + (s + c * (_L * n_s))])
                v = jnp.minimum(jnp.maximum(v, 0), _NUM_EMBEDDINGS - 1)
                idx_v[p][pl.ds(c * _L, _L)] = v
                return carry
            lax.fori_loop(0, bpw // _L, body, 0)

        def fire_gathers(p):
            return [
                pltpu.async_copy(
                    table_hbm.at[idx_v[p].at[pl.ds(j * _IDXW, _IDXW)]],
                    rows_v[p].at[pl.ds(j * _IDXW, _IDXW)],
                    gsem[p])
                for j in range(kk)
            ]

        rowt = [iota + t * _L for t in range(_IDXW // _L)]

        def transpose_fill(p):
            def body(u, carry):
                for rc_off in range(2):
                    rc = 2 * u + rc_off
                    r = rc // kk
                    c = rc % kk
                    cbase = jnp.full((_L,), c * _IDXW, jnp.int32)
                    rows8 = [rv + cbase for rv in rowt]
                    for d0 in range(0, 8, 2):
                        vecs = []
                        for d in (d0, d0 + 1):
                            col = jnp.full((_L,), r * 8 + d, jnp.int32)
                            for t in range(_IDXW // _L):
                                vecs.append(
                                    plsc.load_gather(rows_v[p],
                                                     [rows8[t], col]))
                        for i, vec in enumerate(vecs):
                            d, t = divmod(i, _IDXW // _L)
                            stage_v[p][r, c, d0 + d, pl.ds(t * _L, _L)] = vec
                return carry
            lax.fori_loop(0, n_rt * kk // 2, body, 0)

        def fire_stores(s, p):
            return [
                pltpu.async_copy(
                    stage_v[p].at[r],
                    out_hbm.at[s, r, pl.ds(ct0, kk)],
                    osem[p])
                for r in range(n_rt)
            ]

        def wait_gathers(p):
            for j in range(kk):
                pltpu.make_async_copy(
                    table_hbm.at[idx_v[p].at[pl.ds(j * _IDXW, _IDXW)]],
                    rows_v[p].at[pl.ds(j * _IDXW, _IDXW)],
                    gsem[p]).wait()

        def wait_stores(s, p):
            for r in range(n_rt):
                pltpu.make_async_copy(
                    stage_v[p].at[r],
                    out_hbm.at[s, r, pl.ds(ct0, kk)],
                    osem[p]).wait()

        extract_clamp(0, 0)
        fire_gathers(0)
        extract_clamp(1, 1)
        wait_gathers(0)
        fire_gathers(1)
        transpose_fill(0)
        fire_stores(0, 0)
        extract_clamp(2, 0)
        wait_gathers(1)
        fire_gathers(0)
        transpose_fill(1)
        fire_stores(1, 1)

        def pair_body(t, carry):
            s0 = 2 * t
            s1 = s0 + 1
            extract_clamp(s1, 1)
            wait_gathers(0)
            fire_gathers(1)
            wait_stores(s0 - 2, 0)
            transpose_fill(0)
            fire_stores(s0, 0)
            extract_clamp(jnp.minimum(s0 + 2, n_s - 1), 0)
            wait_gathers(1)
            fire_gathers(0)
            wait_stores(s1 - 2, 1)
            transpose_fill(1)
            fire_stores(s1, 1)
            return carry

        lax.fori_loop(1, n_s // 2, pair_body, 0)

        wait_gathers(0)
        wait_stores(n_s - 2, 0)
        wait_stores(n_s - 1, 1)

    return gather_kernel


def kernel(mood_ids, table):
    n_b, n_s = mood_ids.shape
    ids_flat = mood_ids.astype(jnp.int32).reshape(n_b * n_s)
    out5 = _make_kernel(n_b, n_s)(ids_flat, table)
    out = jnp.transpose(out5, (2, 4, 0, 1, 3)).reshape(n_b, n_s, _EMBED_DIM)
    return out


# speedup vs baseline: 21.2264x; 3.2365x over previous
import functools

import jax
import jax.numpy as jnp
from jax import lax
from jax.experimental import pallas as pl
from jax.experimental.pallas import tpu as pltpu
from jax.experimental.pallas import tpu_sc as plsc

_NUM_MOODS = 100000
_EMBED_DIM = 32
_NUM_EMBEDDINGS = _NUM_MOODS + 1

_L = 16
_NW = 32
_IDXW = 128


def _make_kernel(n_b: int, n_s: int):
    assert n_b % (_NW * _IDXW) == 0
    bpw = n_b // _NW
    kk = bpw // _IDXW
    n_ct = n_b // _IDXW
    n_rt = _EMBED_DIM // 8

    mesh = plsc.VectorSubcoreMesh(core_axis_name="c", subcore_axis_name="s")

    @functools.partial(
        pl.kernel,
        mesh=mesh,
        out_type=jax.ShapeDtypeStruct((n_s, n_rt, n_ct * 8 * _IDXW),
                                      jnp.float32),
        scratch_types=[
            pltpu.VMEM((bpw * n_s,), jnp.int32),
            [pltpu.VMEM((bpw,), jnp.int32)] * 2,
            [pltpu.VMEM((bpw, _EMBED_DIM), jnp.float32)] * 2,
            [pltpu.VMEM((n_rt * kk * 8 * _IDXW,), jnp.float32)] * 2,
            pltpu.SemaphoreType.DMA,
            [pltpu.SemaphoreType.DMA] * 2,
            [pltpu.SemaphoreType.DMA] * 2,
        ],
        compiler_params=pltpu.CompilerParams(
            use_tc_tiling_on_sc=False, needs_layout_passes=False),
    )
    def gather_kernel(ids_hbm, table_hbm, out_hbm, blk, idx_v, rows_v, stage_v,
                      bsem, gsem, osem):
        wid = lax.axis_index("s") * 2 + lax.axis_index("c")
        ct0 = wid * kk

        pltpu.async_copy(
            ids_hbm.at[pl.ds(wid * (bpw * n_s), bpw * n_s)], blk, bsem
        ).wait()

        iota = lax.iota(jnp.int32, _L)
        iota_ns = iota * n_s
        iota_d = iota * _EMBED_DIM

        def extract_clamp(s, p):
            def body(c, carry):
                v = plsc.load_gather(blk, [iota_ns + (s + c * (_L * n_s))])
                v = jnp.minimum(jnp.maximum(v, 0), _NUM_EMBEDDINGS - 1)
                idx_v[p][pl.ds(c * _L, _L)] = v
                return carry
            lax.fori_loop(0, bpw // _L, body, 0)

        def fire_gathers(p):
            return [
                pltpu.async_copy(
                    table_hbm.at[idx_v[p].at[pl.ds(j * _IDXW, _IDXW)]],
                    rows_v[p].at[pl.ds(j * _IDXW, _IDXW)],
                    gsem[p])
                for j in range(kk)
            ]

        rot = [(iota + k) & (_L - 1) for k in range(_L)]
        colv = [[h * _L + rk for rk in rot] for h in range(2)]
        posv = [[(cv >> 3) * (kk * 8 * _IDXW) + (cv & 7) * _IDXW + iota
                 for cv in colv[h]] for h in range(2)]

        def transpose_fill(p):
            def body(m, carry):
                rowvec = iota + m * _L
                base = (m // 8) * (8 * _IDXW) + (m % 8) * _L
                bsplat = jnp.full((_L,), base, jnp.int32)
                for h in range(2):
                    vecs = [plsc.load_gather(rows_v[p], [rowvec, colv[h][k]])
                            for k in range(_L)]
                    targs = [posv[h][k] + bsplat for k in range(_L)]
                    for k in range(_L):
                        plsc.store_scatter(stage_v[p], [targs[k]], vecs[k])
                return carry
            lax.fori_loop(0, bpw // _L, body, 0)

        tile_w = kk * 8 * _IDXW

        def fire_stores(s, p):
            return [
                pltpu.async_copy(
                    stage_v[p].at[pl.ds(r * tile_w, tile_w)],
                    out_hbm.at[s, r, pl.ds(ct0 * (8 * _IDXW), tile_w)],
                    osem[p])
                for r in range(n_rt)
            ]

        def wait_gathers(p):
            for j in range(kk):
                pltpu.make_async_copy(
                    table_hbm.at[idx_v[p].at[pl.ds(j * _IDXW, _IDXW)]],
                    rows_v[p].at[pl.ds(j * _IDXW, _IDXW)],
                    gsem[p]).wait()

        def wait_stores(s, p):
            for r in range(n_rt):
                pltpu.make_async_copy(
                    stage_v[p].at[pl.ds(r * tile_w, tile_w)],
                    out_hbm.at[s, r, pl.ds(ct0 * (8 * _IDXW), tile_w)],
                    osem[p]).wait()

        extract_clamp(0, 0)
        fire_gathers(0)
        extract_clamp(1, 1)
        wait_gathers(0)
        fire_gathers(1)
        transpose_fill(0)
        fire_stores(0, 0)
        extract_clamp(2, 0)
        wait_gathers(1)
        fire_gathers(0)
        transpose_fill(1)
        fire_stores(1, 1)

        def pair_body(t, carry):
            s0 = 2 * t
            s1 = s0 + 1
            extract_clamp(s1, 1)
            wait_gathers(0)
            fire_gathers(1)
            wait_stores(s0 - 2, 0)
            transpose_fill(0)
            fire_stores(s0, 0)
            extract_clamp(jnp.minimum(s0 + 2, n_s - 1), 0)
            wait_gathers(1)
            fire_gathers(0)
            wait_stores(s1 - 2, 1)
            transpose_fill(1)
            fire_stores(s1, 1)
            return carry

        lax.fori_loop(1, n_s // 2, pair_body, 0)

        wait_gathers(0)
        wait_stores(n_s - 2, 0)
        wait_stores(n_s - 1, 1)

    return gather_kernel


def kernel(mood_ids, table):
    n_b, n_s = mood_ids.shape
    ids_flat = mood_ids.astype(jnp.int32).reshape(n_b * n_s)
    out5 = _make_kernel(n_b, n_s)(ids_flat, table)
    out5 = out5.reshape(n_s, _EMBED_DIM // 8, n_b // 128, 8, 128)
    out = jnp.transpose(out5, (2, 4, 0, 1, 3)).reshape(n_b, n_s, _EMBED_DIM)
    return out
